# baseline (speedup 1.0000x reference)
import functools

import jax
import jax.numpy as jnp
from jax import lax
from jax.experimental import pallas as pl
from jax.experimental.pallas import tpu as pltpu
from jax.experimental.pallas import tpu_sc as plsc

N_ATOMS = 100000
N_PAIRS = 6400000
L = 16

PAD_N = 106496

CHUNK = 10000


def _sc_partial_hists(pair_i):
  mesh = plsc.VectorSubcoreMesh(core_axis_name="c", subcore_axis_name="s")
  nw = mesh.num_cores * mesh.num_subcores
  per_w = N_PAIRS // nw
  n_chunks = per_w // CHUNK

  @functools.partial(
      pl.kernel,
      out_type=jax.ShapeDtypeStruct((nw, PAD_N), jnp.int32),
      mesh=mesh,
      compiler_params=pltpu.CompilerParams(needs_layout_passes=False),
      scratch_types=[
          pltpu.VMEM((PAD_N,), jnp.int32),
          pltpu.VMEM((CHUNK + 32,), jnp.int32),
          pltpu.VMEM((CHUNK + 32,), jnp.int32),
          pltpu.SemaphoreType.DMA,
          pltpu.SemaphoreType.DMA,
      ],
  )
  def hist_kernel(x_hbm, out_hbm, hist, buf0, buf1, sem0, sem1):
    wid = lax.axis_index("s") * mesh.num_cores + lax.axis_index("c")
    base = wid * per_w

    bufs = (buf0, buf1)
    sems = (sem0, sem1)
    descs = [None] * n_chunks
    descs[0] = pltpu.async_copy(
        x_hbm.at[pl.ds(base, CHUNK)], buf0.at[pl.ds(16, CHUNK)], sem0)

    zv = jnp.zeros((L,), jnp.int32)

    @plsc.parallel_loop(0, PAD_N // L, unroll=8)
    def _(j):
      hist[pl.ds(j * L, L)] = zv

    iota = lax.iota(jnp.int32, L)
    lane0 = iota == 0
    lane15 = iota == L - 1

    def count_body(buf):
      ones = jnp.ones((L,), jnp.int32)

      @plsc.parallel_loop(0, CHUNK // L, unroll=8)
      def _(i):
        off = i * L + 16
        v = buf[pl.ds(off, L)]
        plsc.addupdate_scatter(hist, [v], ones)

    for k in range(n_chunks):
      if k + 1 < n_chunks:
        descs[k + 1] = pltpu.async_copy(
            x_hbm.at[pl.ds(base + (k + 1) * CHUNK, CHUNK)],
            bufs[(k + 1) % 2].at[pl.ds(16, CHUNK)], sems[(k + 1) % 2])
      descs[k].wait()
      count_body(bufs[k % 2])

    pltpu.sync_copy(hist, out_hbm.at[wid])

  return hist_kernel(pair_i)


def _tc_reduce(partials):
  nw = partials.shape[0]
  blk = 8192
  n_grid = PAD_N // blk

  def reduce_kernel(x_ref, o_ref):
    o_ref[...] = jnp.sum(x_ref[...], axis=0)

  return pl.pallas_call(
      reduce_kernel,
      grid=(n_grid,),
      in_specs=[pl.BlockSpec((nw, blk), lambda i: (0, i))],
      out_specs=pl.BlockSpec((blk,), lambda i: (i,)),
      out_shape=jax.ShapeDtypeStruct((PAD_N,), jnp.int32),
  )(partials)


@jax.jit
def kernel(pair_i):
  pair_i = pair_i.astype(jnp.int32)
  partials = _sc_partial_hists(pair_i)
  return _tc_reduce(partials)[:N_ATOMS]


# speedup vs baseline: 2.0656x; 2.0656x over previous
import functools

import jax
import jax.numpy as jnp
from jax import lax
from jax.experimental import pallas as pl
from jax.experimental.pallas import tpu as pltpu
from jax.experimental.pallas import tpu_sc as plsc

N_ATOMS = 100000
N_PAIRS = 6400000
L = 16

PAD_N = 106496

CHUNK = 10000


def _sc_partial_hists(pair_i):
  mesh = plsc.VectorSubcoreMesh(core_axis_name="c", subcore_axis_name="s")
  nw = mesh.num_cores * mesh.num_subcores
  per_w = N_PAIRS // nw
  n_chunks = per_w // CHUNK

  @functools.partial(
      pl.kernel,
      out_type=jax.ShapeDtypeStruct((nw, PAD_N), jnp.int32),
      mesh=mesh,
      compiler_params=pltpu.CompilerParams(needs_layout_passes=False),
      scratch_types=[
          pltpu.VMEM((PAD_N,), jnp.int32),
          pltpu.VMEM((CHUNK + 32,), jnp.int32),
          pltpu.VMEM((CHUNK + 32,), jnp.int32),
          pltpu.SemaphoreType.DMA,
          pltpu.SemaphoreType.DMA,
      ],
  )
  def hist_kernel(x_hbm, out_hbm, hist, buf0, buf1, sem0, sem1):
    wid = lax.axis_index("s") * mesh.num_cores + lax.axis_index("c")
    base = wid * per_w

    bufs = (buf0, buf1)
    sems = (sem0, sem1)
    descs = [None] * n_chunks
    descs[0] = pltpu.async_copy(
        x_hbm.at[pl.ds(base, CHUNK)], buf0.at[pl.ds(16, CHUNK)], sem0)

    zv = jnp.zeros((L,), jnp.int32)

    @plsc.parallel_loop(0, PAD_N // L, unroll=8)
    def _(j):
      hist[pl.ds(j * L, L)] = zv

    iota = lax.iota(jnp.int32, L)
    lane0 = iota == 0
    lane15 = iota == L - 1

    def count_body(buf):
      @plsc.parallel_loop(0, CHUNK // L, unroll=8)
      def _(i):
        off = i * L + 16
        v = buf[pl.ds(off, L)]
        prv = buf[pl.ds(off - 1, L)]
        nxt = buf[pl.ds(off + 1, L)]
        m_start = (v != prv) | lane0
        m_end = (v != nxt) | lane15
        s = plsc.cummax(jnp.where(m_start, iota, 0))
        cnt = iota - s + 1
        plsc.addupdate_scatter(hist, [v], cnt, mask=m_end)

    for k in range(n_chunks):
      if k + 1 < n_chunks:
        descs[k + 1] = pltpu.async_copy(
            x_hbm.at[pl.ds(base + (k + 1) * CHUNK, CHUNK)],
            bufs[(k + 1) % 2].at[pl.ds(16, CHUNK)], sems[(k + 1) % 2])
      descs[k].wait()
      count_body(bufs[k % 2])

    pltpu.sync_copy(hist, out_hbm.at[wid])

  return hist_kernel(pair_i)


def _tc_reduce(partials):
  nw = partials.shape[0]
  blk = 8192
  n_grid = PAD_N // blk

  def reduce_kernel(x_ref, o_ref):
    o_ref[...] = jnp.sum(x_ref[...], axis=0)

  return pl.pallas_call(
      reduce_kernel,
      grid=(n_grid,),
      in_specs=[pl.BlockSpec((nw, blk), lambda i: (0, i))],
      out_specs=pl.BlockSpec((blk,), lambda i: (i,)),
      out_shape=jax.ShapeDtypeStruct((PAD_N,), jnp.int32),
  )(partials)


@jax.jit
def kernel(pair_i):
  pair_i = pair_i.astype(jnp.int32)
  partials = _sc_partial_hists(pair_i)
  return _tc_reduce(partials)[:N_ATOMS]


# speedup vs baseline: 2.1135x; 1.0232x over previous
import functools

import jax
import jax.numpy as jnp
from jax import lax
from jax.experimental import pallas as pl
from jax.experimental.pallas import tpu as pltpu
from jax.experimental.pallas import tpu_sc as plsc

N_ATOMS = 100000
N_PAIRS = 6400000
L = 16

PAD_N = 106496
ROWS = PAD_N // L
RB = 64

CHUNK = 8000


def _sc_hists(pair_i):
  mesh = plsc.VectorSubcoreMesh(core_axis_name="c", subcore_axis_name="s")
  nc, ns = mesh.num_cores, mesh.num_subcores
  nw = nc * ns
  per_w = N_PAIRS // nw
  n_chunks = per_w // CHUNK
  stripe = ROWS // ns

  @functools.partial(
      pl.kernel,
      out_type=jax.ShapeDtypeStruct((nc, ROWS, L), jnp.int32),
      mesh=mesh,
      compiler_params=pltpu.CompilerParams(
          needs_layout_passes=False, use_tc_tiling_on_sc=False),
      scratch_types=[
          pltpu.VMEM((ROWS, L), jnp.int32),
          pltpu.VMEM((CHUNK + 32,), jnp.int32),
          pltpu.VMEM((CHUNK + 32,), jnp.int32),
          pltpu.VMEM((RB,), jnp.int32),
          pltpu.VMEM((L,), jnp.int32),
          pltpu.VMEM((L,), jnp.int32),
          pltpu.VMEM_SHARED((ROWS, L), jnp.int32),
          pltpu.SemaphoreType.DMA,
          pltpu.SemaphoreType.DMA,
      ],
  )
  def hist_kernel(x_hbm, out_hbm, hist, buf0, buf1, idx, first, last,
                  shared, sem0, sem1):
    cid = lax.axis_index("c")
    sid = lax.axis_index("s")
    wid = sid * nc + cid
    base = wid * per_w

    bufs = (buf0, buf1)
    sems = (sem0, sem1)
    descs = [None] * n_chunks
    descs[0] = pltpu.async_copy(
        x_hbm.at[pl.ds(base, CHUNK)], buf0.at[pl.ds(16, CHUNK)], sem0)

    pltpu.sync_copy(x_hbm.at[pl.ds(base, L)], first)
    pltpu.sync_copy(x_hbm.at[pl.ds(base + per_w - L, L)], last)
    lo = jnp.min(first[...])
    hi = jnp.max(last[...])
    blk_lo = lo >> 10
    blk_hi = (hi >> 10) + 1

    zv = jnp.zeros((L,), jnp.int32)

    @plsc.parallel_loop(0, stripe, unroll=8)
    def _(j):
      hist[j, :] = zv

    pltpu.sync_copy(hist.at[pl.ds(0, stripe)],
                    shared.at[pl.ds(sid * stripe, stripe)])

    def zero_body(j, c):
      hist[j, :] = zv
      return c

    lax.fori_loop(blk_lo * RB, blk_hi * RB, zero_body, 0)

    iota = lax.iota(jnp.int32, L)
    lane0 = iota == 0
    lane15 = iota == L - 1

    def count_body(buf):
      @plsc.parallel_loop(0, CHUNK // L, unroll=8)
      def _(i):
        off = i * L + 16
        v = buf[pl.ds(off, L)]
        prv = buf[pl.ds(off - 1, L)]
        nxt = buf[pl.ds(off + 1, L)]
        m_start = (v != prv) | lane0
        m_end = (v != nxt) | lane15
        s = plsc.cummax(jnp.where(m_start, iota, 0))
        cnt = iota - s + 1
        plsc.addupdate_scatter(hist, [v >> 4, v & (L - 1)], cnt, mask=m_end)

    for k in range(n_chunks):
      if k + 1 < n_chunks:
        descs[k + 1] = pltpu.async_copy(
            x_hbm.at[pl.ds(base + (k + 1) * CHUNK, CHUNK)],
            bufs[(k + 1) % 2].at[pl.ds(16, CHUNK)], sems[(k + 1) % 2])
      descs[k].wait()
      count_body(bufs[k % 2])

    plsc.subcore_barrier()

    def merge_body(b, c):
      rb = b * RB
      for j in range(RB // L):
        idx[pl.ds(j * L, L)] = rb + j * L + iota
      pltpu.sync_copy(hist.at[pl.ds(rb, RB)], shared.at[idx], add=True)
      return c

    lax.fori_loop(blk_lo, blk_hi, merge_body, 0)

    plsc.subcore_barrier()

    pltpu.sync_copy(shared.at[pl.ds(sid * stripe, stripe)],
                    out_hbm.at[cid].at[pl.ds(sid * stripe, stripe)])

  return hist_kernel(pair_i)


def _tc_reduce(partials):
  ncores = partials.shape[0]
  blk = 8192
  n_grid = PAD_N // blk

  def reduce_kernel(x_ref, o_ref):
    o_ref[...] = jnp.sum(x_ref[...], axis=0)

  return pl.pallas_call(
      reduce_kernel,
      grid=(n_grid,),
      in_specs=[pl.BlockSpec((ncores, blk), lambda i: (0, i))],
      out_specs=pl.BlockSpec((blk,), lambda i: (i,)),
      out_shape=jax.ShapeDtypeStruct((PAD_N,), jnp.int32),
  )(partials)


@jax.jit
def kernel(pair_i):
  pair_i = pair_i.astype(jnp.int32)
  partials = _sc_hists(pair_i).reshape(2, PAD_N)
  return _tc_reduce(partials)[:N_ATOMS]
